# baseline (speedup 1.0000x reference)
import functools

import jax
import jax.numpy as jnp
from jax.experimental import pallas as pl

MAX_SEQ = 512
INF = -1000000000.0


def _mask_body(src_stop, tgt_stop, enc_ref, dec_self_ref, cross_ref):
    row = jax.lax.broadcasted_iota(jnp.int32, (MAX_SEQ, MAX_SEQ), 0)
    col = jax.lax.broadcasted_iota(jnp.int32, (MAX_SEQ, MAX_SEQ), 1)
    src_row = row >= src_stop
    src_col = col >= src_stop
    tgt_row = row >= tgt_stop
    tgt_col = col >= tgt_stop
    look_ahead = col > row
    zero = jnp.zeros((MAX_SEQ, MAX_SEQ), jnp.float32)
    inf = jnp.full((MAX_SEQ, MAX_SEQ), INF, jnp.float32)
    enc_ref[0] = jnp.where(src_row | src_col, inf, zero)
    dec_self_ref[0] = jnp.where(look_ahead | tgt_row | tgt_col, inf, zero)
    cross_ref[0] = jnp.where(src_col | tgt_row, inf, zero)


def kernel(source_batch, target_batch):
    num_sentences = source_batch.shape[0]
    src_stop = source_batch.shape[1] + 1
    tgt_stop = target_batch.shape[1] + 1

    out_shape = jax.ShapeDtypeStruct((num_sentences, MAX_SEQ, MAX_SEQ),
                                     jnp.float32)
    grid = (num_sentences,)
    spec = pl.BlockSpec((1, MAX_SEQ, MAX_SEQ), lambda i: (i, 0, 0))
    enc, dec_self, cross = pl.pallas_call(
        functools.partial(_mask_body, src_stop, tgt_stop),
        grid=grid,
        out_specs=(spec, spec, spec),
        out_shape=(out_shape, out_shape, out_shape),
    )()
    return enc, dec_self, cross
